# baseline (device time: 106391 ns/iter reference)
import jax
import jax.numpy as jnp
from jax import lax
from jax.experimental import pallas as pl
from jax.experimental.pallas import tpu as pltpu

N_DEV = 8
B_PER = 2
HQ_PER = 4
SQ = 256
SKV = 256
DH = 64
D_MODEL = 512
HK = HQ_PER * DH

_SCHED = [
    [(0, 0)],
    [(0, 1), (1, 0), (3, 0)],
    [(2, 0), (1, 1), (3, 1)],
    [(2, 1)],
]


def kernel(x, Wq, K_ext, V_ext, Wo):
    wq_bf = (Wq * 0.125).astype(jnp.bfloat16)
    wo_bf = Wo.T.astype(jnp.bfloat16)
    w_pack = jnp.stack([wq_bf, wo_bf])
    k2 = K_ext.reshape(2 * N_DEV, SKV, N_DEV * HK)
    v2 = V_ext.reshape(2 * N_DEV, SKV, N_DEV * HK)

    def body(x_ref, w_ref, k_hbm, v_hbm, out_ref,
             comm, k_blk, v_blk, ksem, vsem,
             z_send, z_recv, cw_send, cw_recv, ccw_send, ccw_recv):
        my = lax.axis_index("i")
        b0 = my * B_PER

        base = jnp.where(my < 4, 0, 4)
        p = my - base
        right = base + lax.rem(p + 1, 4)
        left = base + lax.rem(p + 3, 4)
        zp = lax.rem(my + 4, N_DEV)

        def origin_of(j, zb):
            o = base + lax.rem(p - j + 4, 4)
            return lax.rem(o + 4 * zb, N_DEV)

        comm[0, 0] = w_ref[...]

        barrier = pltpu.get_barrier_semaphore()
        for nbr in (left, right, zp):
            pl.semaphore_signal(barrier, inc=1, device_id=(nbr,),
                                device_id_type=pl.DeviceIdType.MESH)
        pl.semaphore_wait(barrier, 3)

        qi = lax.broadcasted_iota(jnp.int32, (SQ, SKV), 0)
        ki = lax.broadcasted_iota(jnp.int32, (SQ, SKV), 1)
        mask = (jnp.abs(qi - ki) <= 128) | (ki < 32) | (qi < 32)
        madd = jnp.where(mask, 0.0, -1e9).astype(jnp.float32)

        x2_bf = x_ref[...].reshape(B_PER * SQ, D_MODEL).astype(jnp.bfloat16)

        def rdma(src_jz, dst_jz, dev, ssem, rsem):
            return pltpu.make_async_remote_copy(
                src_ref=comm.at[src_jz[0], src_jz[1]],
                dst_ref=comm.at[dst_jz[0], dst_jz[1]],
                send_sem=ssem, recv_sem=rsem,
                device_id=(dev,), device_id_type=pl.DeviceIdType.MESH)

        def start_kv(r):
            buf = r % 2
            copies = []
            for i, (j, zb) in enumerate(_SCHED[r]):
                origin = origin_of(j, zb)
                ck = pltpu.make_async_copy(
                    k_hbm.at[pl.ds(b0, B_PER), :, pl.ds(origin * HK, HK)],
                    k_blk.at[buf, i], ksem.at[buf, i])
                cv = pltpu.make_async_copy(
                    v_hbm.at[pl.ds(b0, B_PER), :, pl.ds(origin * HK, HK)],
                    v_blk.at[buf, i], vsem.at[buf, i])
                ck.start()
                cv.start()
                copies += [ck, cv]
            return copies

        def compute(jz, buf, oi, first):
            wq_o = comm[jz[0], jz[1], 0]
            wo_o = comm[jz[0], jz[1], 1]
            q2_bf = lax.dot_general(
                x2_bf, wq_o, (((1,), (0,)), ((), ())),
                preferred_element_type=jnp.float32
            ).astype(jnp.bfloat16)
            kst = k_blk[buf, oi].reshape(B_PER * SKV, HK).astype(jnp.bfloat16)
            vst = v_blk[buf, oi].reshape(B_PER * SKV, HK).astype(jnp.bfloat16)
            ctx_h = []
            for hh in range(HQ_PER):
                q_h = q2_bf[:, hh * DH:(hh + 1) * DH]
                k_h = kst[:, hh * DH:(hh + 1) * DH]
                s_full = lax.dot_general(
                    q_h, k_h, (((1,), (1,)), ((), ())),
                    preferred_element_type=jnp.float32)
                v_h = vst[:, hh * DH:(hh + 1) * DH]
                cb = []
                for b in range(B_PER):
                    s = s_full[b * SQ:(b + 1) * SQ,
                               b * SKV:(b + 1) * SKV] + madd
                    e = jnp.exp(s)
                    w = (e * (1.0 / jnp.sum(e, axis=1, keepdims=True))
                         ).astype(jnp.bfloat16)
                    cb.append(lax.dot_general(
                        w, v_h[b * SKV:(b + 1) * SKV],
                        (((1,), (0,)), ((), ())),
                        preferred_element_type=jnp.float32))
                ctx_h.append(jnp.concatenate(cb, axis=0))
            ctx2 = jnp.concatenate(ctx_h, axis=1).astype(jnp.bfloat16)
            contrib = lax.dot_general(
                ctx2, wo_o, (((1,), (1,)), ((), ())),
                preferred_element_type=jnp.float32)
            contrib = contrib.reshape(B_PER, SQ, D_MODEL)
            if first:
                out_ref[...] = contrib
            else:
                out_ref[...] = out_ref[...] + contrib

        kv_pending = start_kv(0)

        sched_rdmas = [
            [rdma((0, 0), (0, 1), zp, z_send, z_recv),
             rdma((0, 0), (1, 0), right, cw_send.at[0], cw_recv.at[0]),
             rdma((0, 0), (3, 0), left, ccw_send.at[0], ccw_recv.at[0])],
            [rdma((1, 0), (2, 0), right, cw_send.at[1], cw_recv.at[1]),
             rdma((0, 1), (1, 1), right, cw_send.at[2], cw_recv.at[2]),
             rdma((0, 1), (3, 1), left, ccw_send.at[1], ccw_recv.at[1])],
            [rdma((1, 1), (2, 1), right, cw_send.at[3], cw_recv.at[3])],
            [],
        ]

        for r in range(4):
            for rd in sched_rdmas[r]:
                rd.start()
            if r < 3:
                next_kv = start_kv(r + 1)

            for c in kv_pending:
                c.wait()

            for i, jz in enumerate(_SCHED[r]):
                compute(jz, r % 2, i, first=(r == 0))

            for rd in sched_rdmas[r]:
                rd.wait()
            if r < 3:
                kv_pending = next_kv

    return pl.pallas_call(
        body,
        out_shape=jax.ShapeDtypeStruct((B_PER, SQ, D_MODEL), jnp.float32),
        in_specs=[
            pl.BlockSpec(memory_space=pltpu.MemorySpace.VMEM),
            pl.BlockSpec(memory_space=pltpu.MemorySpace.VMEM),
            pl.BlockSpec(memory_space=pl.ANY),
            pl.BlockSpec(memory_space=pl.ANY),
        ],
        out_specs=pl.BlockSpec(memory_space=pltpu.MemorySpace.VMEM),
        scratch_shapes=[
            pltpu.MemorySpace.VMEM((4, 2, 2, D_MODEL, HK),
                                   jnp.bfloat16),
            pltpu.MemorySpace.VMEM((2, 3, B_PER, SKV, HK),
                                   jnp.float32),
            pltpu.MemorySpace.VMEM((2, 3, B_PER, SKV, HK),
                                   jnp.float32),
            pltpu.SemaphoreType.DMA((2, 3)),
            pltpu.SemaphoreType.DMA((2, 3)),
            pltpu.SemaphoreType.DMA,
            pltpu.SemaphoreType.DMA,
            pltpu.SemaphoreType.DMA((4,)),
            pltpu.SemaphoreType.DMA((4,)),
            pltpu.SemaphoreType.DMA((2,)),
            pltpu.SemaphoreType.DMA((2,)),
        ],
        compiler_params=pltpu.CompilerParams(collective_id=0),
    )(x, w_pack, k2, v2)
